# baseline (device time: 47190 ns/iter reference)
import jax
import jax.numpy as jnp
from jax import lax
from jax.experimental import pallas as pl
from jax.experimental.pallas import tpu as pltpu

T = 1024
D = 1024
F = 2048
E = 4
E_LOCAL = 2
GC = 288
NCH = 4
HC = E_LOCAL * GC // NCH

CONTRACT0 = (((0,), (0,)), ((), ()))
CONTRACT1 = (((1,), (0,)), ((), ()))


def kernel(x, assign, W1, W2):
    a2d = assign.reshape(T, 1)

    def body(x_ref, a_ref, w1_ref, w2_ref, out_ref,
             bufs_ref, y_ref, recvb, pbuf, w1f, w2f, sel_ref,
             send_sems, recv_sems, wcopy_sems):
        my_x = lax.axis_index("x")
        my_y = lax.axis_index("y")
        my_z = lax.axis_index("z")
        peer = (my_x, my_y, 1 - my_z)

        wcopies = []
        for el in range(E_LOCAL):
            c1 = pltpu.make_async_copy(w1_ref.at[el], w1f.at[el],
                                       wcopy_sems.at[el])
            c2 = pltpu.make_async_copy(w2_ref.at[el], w2f.at[el],
                                       wcopy_sems.at[E_LOCAL + el])
            c1.start()
            c2.start()
            wcopies.append((c1, c2))

        barrier = pltpu.get_barrier_semaphore()
        pl.semaphore_signal(barrier, inc=1, device_id=peer,
                            device_id_type=pl.DeviceIdType.MESH)
        pl.semaphore_wait(barrier, 1)

        a = a_ref[:, :]
        onehot = (a == lax.broadcasted_iota(jnp.int32, (T, E), 1))
        onehot = onehot.astype(jnp.float32)
        B = 128
        lower = (lax.broadcasted_iota(jnp.int32, (B, B), 1)
                 < lax.broadcasted_iota(jnp.int32, (B, B), 0))
        lower = lower.astype(jnp.float32)
        carry = jnp.zeros((1, E), jnp.float32)
        cum_blocks = []
        for b in range(T // B):
            oh_b = onehot[b * B:(b + 1) * B, :]
            cum_blocks.append(
                jnp.dot(lower, oh_b, preferred_element_type=jnp.float32)
                + carry)
            carry = carry + jnp.sum(oh_b, axis=0, keepdims=True)
        cum = jnp.concatenate(cum_blocks, axis=0)
        rank = jnp.sum(cum * onehot, axis=1, keepdims=True).astype(jnp.int32)

        key = jnp.where(rank < GC, a * GC + rank, E * GC)
        sel = (lax.broadcasted_iota(jnp.int32, (E * GC, T), 0)
               == key.reshape(1, T)).astype(jnp.bfloat16)
        sel_ref[:, :] = sel
        bufs_ref[:, :] = lax.dot_general(
            sel, x_ref[:, :], dimension_numbers=CONTRACT1,
            preferred_element_type=jnp.float32).astype(jnp.bfloat16)

        peer_base = E_LOCAL * (1 - my_z) * GC
        own_base = E_LOCAL * my_z * GC
        rs = []
        for c in range(NCH):
            r = pltpu.make_async_remote_copy(
                src_ref=bufs_ref.at[pl.ds(peer_base + c * HC, HC)],
                dst_ref=recvb.at[c],
                send_sem=send_sems.at[c], recv_sem=recv_sems.at[c],
                device_id=peer, device_id_type=pl.DeviceIdType.MESH)
            r.start()
            rs.append(r)

        def ffn(xt, el):
            h = lax.dot_general(xt, w1f[el], dimension_numbers=CONTRACT1,
                                preferred_element_type=jnp.float32)
            h = jnp.maximum(h, 0.0).astype(jnp.bfloat16)
            return lax.dot_general(h, w2f[el], dimension_numbers=CONTRACT1,
                                   preferred_element_type=jnp.float32)

        def own_ffn(el):
            y_ref[pl.ds(own_base + el * GC, GC)] = ffn(
                bufs_ref[pl.ds(own_base + el * GC, GC)], el
            ).astype(jnp.bfloat16)

        def recv_ffn(c):
            rs[c].wait_recv()
            pbuf[c] = ffn(recvb[c], c // (NCH // E_LOCAL)).astype(jnp.bfloat16)
            rr = pltpu.make_async_remote_copy(
                src_ref=pbuf.at[c],
                dst_ref=y_ref.at[pl.ds(own_base + c * HC, HC)],
                send_sem=send_sems.at[NCH + c],
                recv_sem=recv_sems.at[NCH + c],
                device_id=peer, device_id_type=pl.DeviceIdType.MESH)
            rr.start()
            rets.append(rr)

        rets = []
        wcopies[0][0].wait()
        wcopies[0][1].wait()
        own_ffn(0)
        recv_ffn(0)
        recv_ffn(1)
        wcopies[1][0].wait()
        wcopies[1][1].wait()
        recv_ffn(2)
        recv_ffn(3)
        own_ffn(1)

        acc = lax.dot_general(
            sel_ref[pl.ds(own_base, E_LOCAL * GC)],
            y_ref[pl.ds(own_base, E_LOCAL * GC)],
            dimension_numbers=CONTRACT0,
            preferred_element_type=jnp.float32)

        for rr in rets:
            rr.wait_recv()

        out_ref[:, :] = acc + lax.dot_general(
            sel_ref[pl.ds(peer_base, E_LOCAL * GC)],
            y_ref[pl.ds(peer_base, E_LOCAL * GC)],
            dimension_numbers=CONTRACT0,
            preferred_element_type=jnp.float32)

        for r in rs + rets:
            r.wait_send()

    return pl.pallas_call(
        body,
        out_shape=jax.ShapeDtypeStruct((T, D), jnp.float32),
        in_specs=[
            pl.BlockSpec(memory_space=pltpu.VMEM),
            pl.BlockSpec(memory_space=pltpu.VMEM),
            pl.BlockSpec(memory_space=pl.ANY),
            pl.BlockSpec(memory_space=pl.ANY),
        ],
        out_specs=pl.BlockSpec(memory_space=pltpu.VMEM),
        scratch_shapes=[
            pltpu.VMEM((E * GC, D), jnp.bfloat16),
            pltpu.VMEM((E * GC, D), jnp.bfloat16),
            pltpu.VMEM((NCH, HC, D), jnp.bfloat16),
            pltpu.VMEM((NCH, HC, D), jnp.bfloat16),
            pltpu.VMEM((E_LOCAL, D, F), jnp.float32),
            pltpu.VMEM((E_LOCAL, F, D), jnp.float32),
            pltpu.VMEM((E * GC, T), jnp.bfloat16),
            pltpu.SemaphoreType.DMA((2 * NCH,)),
            pltpu.SemaphoreType.DMA((2 * NCH,)),
            pltpu.SemaphoreType.DMA((2 * E_LOCAL,)),
        ],
        compiler_params=pltpu.CompilerParams(
            collective_id=0, vmem_limit_bytes=60 * 1024 * 1024),
    )(x, a2d, W1, W2)


# device time: 45942 ns/iter; 1.0272x vs baseline; 1.0272x over previous
import jax
import jax.numpy as jnp
from jax import lax
from jax.experimental import pallas as pl
from jax.experimental.pallas import tpu as pltpu

T = 1024
D = 1024
F = 2048
E = 4
E_LOCAL = 2
GC = 288
NCH = 4
HC = E_LOCAL * GC // NCH

CONTRACT0 = (((0,), (0,)), ((), ()))
CONTRACT1 = (((1,), (0,)), ((), ()))


def kernel(x, assign, W1, W2):
    a2d = assign.reshape(T, 1)

    def body(x_ref, a_ref, w1_ref, w2_ref, out_ref,
             bufs_ref, y_ref, recvb, pbuf, w1f, w2f, sel_ref,
             send_sems, recv_sems, wcopy_sems):
        my_x = lax.axis_index("x")
        my_y = lax.axis_index("y")
        my_z = lax.axis_index("z")
        peer = (my_x, my_y, 1 - my_z)

        wcopies = []
        for el in range(E_LOCAL):
            c1 = pltpu.make_async_copy(w1_ref.at[el], w1f.at[el],
                                       wcopy_sems.at[el])
            c2 = pltpu.make_async_copy(w2_ref.at[el], w2f.at[el],
                                       wcopy_sems.at[E_LOCAL + el])
            c1.start()
            c2.start()
            wcopies.append((c1, c2))

        barrier = pltpu.get_barrier_semaphore()
        pl.semaphore_signal(barrier, inc=1, device_id=peer,
                            device_id_type=pl.DeviceIdType.MESH)
        pl.semaphore_wait(barrier, 1)

        a = a_ref[:, :]
        onehot = (a == lax.broadcasted_iota(jnp.int32, (T, E), 1))
        onehot = onehot.astype(jnp.float32)
        B = 128
        lower = (lax.broadcasted_iota(jnp.int32, (B, B), 1)
                 < lax.broadcasted_iota(jnp.int32, (B, B), 0))
        lower = lower.astype(jnp.float32)
        carry = jnp.zeros((1, E), jnp.float32)
        cum_blocks = []
        for b in range(T // B):
            oh_b = onehot[b * B:(b + 1) * B, :]
            cum_blocks.append(
                jnp.dot(lower, oh_b, preferred_element_type=jnp.float32)
                + carry)
            carry = carry + jnp.sum(oh_b, axis=0, keepdims=True)
        cum = jnp.concatenate(cum_blocks, axis=0)
        rank = jnp.sum(cum * onehot, axis=1, keepdims=True).astype(jnp.int32)

        key = jnp.where(rank < GC, a * GC + rank, E * GC)
        sel = (lax.broadcasted_iota(jnp.int32, (E * GC, T), 0)
               == key.reshape(1, T)).astype(jnp.bfloat16)
        sel_ref[:, :] = sel

        peer_base = E_LOCAL * (1 - my_z) * GC
        own_base = E_LOCAL * my_z * GC
        half = E_LOCAL * GC

        bufs_ref[pl.ds(peer_base, half)] = lax.dot_general(
            sel_ref[pl.ds(peer_base, half)], x_ref[:, :],
            dimension_numbers=CONTRACT1,
            preferred_element_type=jnp.float32).astype(jnp.bfloat16)

        rs = []
        for c in range(NCH):
            r = pltpu.make_async_remote_copy(
                src_ref=bufs_ref.at[pl.ds(peer_base + c * HC, HC)],
                dst_ref=recvb.at[c],
                send_sem=send_sems.at[c], recv_sem=recv_sems.at[c],
                device_id=peer, device_id_type=pl.DeviceIdType.MESH)
            r.start()
            rs.append(r)

        bufs_ref[pl.ds(own_base, half)] = lax.dot_general(
            sel_ref[pl.ds(own_base, half)], x_ref[:, :],
            dimension_numbers=CONTRACT1,
            preferred_element_type=jnp.float32).astype(jnp.bfloat16)

        def ffn(xt, el):
            h = lax.dot_general(xt, w1f[el], dimension_numbers=CONTRACT1,
                                preferred_element_type=jnp.float32)
            h = jnp.maximum(h, 0.0).astype(jnp.bfloat16)
            return lax.dot_general(h, w2f[el], dimension_numbers=CONTRACT1,
                                   preferred_element_type=jnp.float32)

        def own_ffn(el):
            y_ref[pl.ds(own_base + el * GC, GC)] = ffn(
                bufs_ref[pl.ds(own_base + el * GC, GC)], el
            ).astype(jnp.bfloat16)

        def recv_ffn(c, splits=1):
            rs[c].wait_recv()
            el = c // (NCH // E_LOCAL)
            sc = HC // splits
            for s in range(splits):
                pbuf[c, pl.ds(s * sc, sc)] = ffn(
                    recvb[c, pl.ds(s * sc, sc)], el).astype(jnp.bfloat16)
                rr = pltpu.make_async_remote_copy(
                    src_ref=pbuf.at[c, pl.ds(s * sc, sc)],
                    dst_ref=y_ref.at[pl.ds(own_base + c * HC + s * sc, sc)],
                    send_sem=send_sems.at[NCH + len(rets)],
                    recv_sem=recv_sems.at[NCH + len(rets)],
                    device_id=peer, device_id_type=pl.DeviceIdType.MESH)
                rr.start()
                rets.append(rr)

        rets = []
        wcopies[0][0].wait()
        wcopies[0][1].wait()
        own_ffn(0)
        recv_ffn(0)
        recv_ffn(1)
        wcopies[1][0].wait()
        wcopies[1][1].wait()
        recv_ffn(2)
        recv_ffn(3, splits=2)
        own_ffn(1)

        acc = lax.dot_general(
            sel_ref[pl.ds(own_base, E_LOCAL * GC)],
            y_ref[pl.ds(own_base, E_LOCAL * GC)],
            dimension_numbers=CONTRACT0,
            preferred_element_type=jnp.float32)

        for rr in rets:
            rr.wait_recv()

        out_ref[:, :] = acc + lax.dot_general(
            sel_ref[pl.ds(peer_base, E_LOCAL * GC)],
            y_ref[pl.ds(peer_base, E_LOCAL * GC)],
            dimension_numbers=CONTRACT0,
            preferred_element_type=jnp.float32)

        for r in rs + rets:
            r.wait_send()

    return pl.pallas_call(
        body,
        out_shape=jax.ShapeDtypeStruct((T, D), jnp.float32),
        in_specs=[
            pl.BlockSpec(memory_space=pltpu.VMEM),
            pl.BlockSpec(memory_space=pltpu.VMEM),
            pl.BlockSpec(memory_space=pl.ANY),
            pl.BlockSpec(memory_space=pl.ANY),
        ],
        out_specs=pl.BlockSpec(memory_space=pltpu.VMEM),
        scratch_shapes=[
            pltpu.VMEM((E * GC, D), jnp.bfloat16),
            pltpu.VMEM((E * GC, D), jnp.bfloat16),
            pltpu.VMEM((NCH, HC, D), jnp.bfloat16),
            pltpu.VMEM((NCH, HC, D), jnp.bfloat16),
            pltpu.VMEM((E_LOCAL, D, F), jnp.float32),
            pltpu.VMEM((E_LOCAL, F, D), jnp.float32),
            pltpu.VMEM((E * GC, T), jnp.bfloat16),
            pltpu.SemaphoreType.DMA((2 * NCH + 1,)),
            pltpu.SemaphoreType.DMA((2 * NCH + 1,)),
            pltpu.SemaphoreType.DMA((2 * E_LOCAL,)),
        ],
        compiler_params=pltpu.CompilerParams(
            collective_id=0, vmem_limit_bytes=60 * 1024 * 1024),
    )(x, a2d, W1, W2)


# device time: 45911 ns/iter; 1.0279x vs baseline; 1.0007x over previous
import jax
import jax.numpy as jnp
from jax import lax
from jax.experimental import pallas as pl
from jax.experimental.pallas import tpu as pltpu

T = 1024
D = 1024
F = 2048
E = 4
E_LOCAL = 2
GC = 288
NCH = 4
HC = E_LOCAL * GC // NCH

CONTRACT0 = (((0,), (0,)), ((), ()))
CONTRACT1 = (((1,), (0,)), ((), ()))


def kernel(x, assign, W1, W2):
    a2d = assign.reshape(T, 1)

    def body(x_ref, a_ref, w1_ref, w2_ref, out_ref,
             bufs_ref, y_ref, recvb, pbuf, w1f, w2f, sel_ref, obuf,
             send_sems, recv_sems, wcopy_sems, out_sem):
        my_x = lax.axis_index("x")
        my_y = lax.axis_index("y")
        my_z = lax.axis_index("z")
        peer = (my_x, my_y, 1 - my_z)

        wcopies = []
        for el in range(E_LOCAL):
            c1 = pltpu.make_async_copy(w1_ref.at[el], w1f.at[el],
                                       wcopy_sems.at[el])
            c2 = pltpu.make_async_copy(w2_ref.at[el], w2f.at[el],
                                       wcopy_sems.at[E_LOCAL + el])
            c1.start()
            c2.start()
            wcopies.append((c1, c2))

        barrier = pltpu.get_barrier_semaphore()
        pl.semaphore_signal(barrier, inc=1, device_id=peer,
                            device_id_type=pl.DeviceIdType.MESH)
        pl.semaphore_wait(barrier, 1)

        a = a_ref[:, :]
        onehot = (a == lax.broadcasted_iota(jnp.int32, (T, E), 1))
        onehot = onehot.astype(jnp.float32)
        B = 128
        lower = (lax.broadcasted_iota(jnp.int32, (B, B), 1)
                 < lax.broadcasted_iota(jnp.int32, (B, B), 0))
        lower = lower.astype(jnp.float32)
        carry = jnp.zeros((1, E), jnp.float32)
        cum_blocks = []
        for b in range(T // B):
            oh_b = onehot[b * B:(b + 1) * B, :]
            cum_blocks.append(
                jnp.dot(lower, oh_b, preferred_element_type=jnp.float32)
                + carry)
            carry = carry + jnp.sum(oh_b, axis=0, keepdims=True)
        cum = jnp.concatenate(cum_blocks, axis=0)
        rank = jnp.sum(cum * onehot, axis=1, keepdims=True).astype(jnp.int32)

        key = jnp.where(rank < GC, a * GC + rank, E * GC)
        sel = (lax.broadcasted_iota(jnp.int32, (E * GC, T), 0)
               == key.reshape(1, T)).astype(jnp.bfloat16)
        sel_ref[:, :] = sel

        peer_base = E_LOCAL * (1 - my_z) * GC
        own_base = E_LOCAL * my_z * GC
        half = E_LOCAL * GC

        bufs_ref[pl.ds(peer_base, half)] = lax.dot_general(
            sel_ref[pl.ds(peer_base, half)], x_ref[:, :],
            dimension_numbers=CONTRACT1,
            preferred_element_type=jnp.float32).astype(jnp.bfloat16)

        rs = []
        for c in range(NCH):
            r = pltpu.make_async_remote_copy(
                src_ref=bufs_ref.at[pl.ds(peer_base + c * HC, HC)],
                dst_ref=recvb.at[c],
                send_sem=send_sems.at[c], recv_sem=recv_sems.at[c],
                device_id=peer, device_id_type=pl.DeviceIdType.MESH)
            r.start()
            rs.append(r)

        bufs_ref[pl.ds(own_base, half)] = lax.dot_general(
            sel_ref[pl.ds(own_base, half)], x_ref[:, :],
            dimension_numbers=CONTRACT1,
            preferred_element_type=jnp.float32).astype(jnp.bfloat16)

        def ffn(xt, el):
            h = lax.dot_general(xt, w1f[el], dimension_numbers=CONTRACT1,
                                preferred_element_type=jnp.float32)
            h = jnp.maximum(h, 0.0).astype(jnp.bfloat16)
            return lax.dot_general(h, w2f[el], dimension_numbers=CONTRACT1,
                                   preferred_element_type=jnp.float32)

        def own_ffn(el):
            y_ref[pl.ds(own_base + el * GC, GC)] = ffn(
                bufs_ref[pl.ds(own_base + el * GC, GC)], el
            ).astype(jnp.bfloat16)

        def recv_ffn(c, splits=1):
            rs[c].wait_recv()
            el = c // (NCH // E_LOCAL)
            sc = HC // splits
            for s in range(splits):
                pbuf[c, pl.ds(s * sc, sc)] = ffn(
                    recvb[c, pl.ds(s * sc, sc)], el).astype(jnp.bfloat16)
                rr = pltpu.make_async_remote_copy(
                    src_ref=pbuf.at[c, pl.ds(s * sc, sc)],
                    dst_ref=y_ref.at[pl.ds(own_base + c * HC + s * sc, sc)],
                    send_sem=send_sems.at[NCH + len(rets)],
                    recv_sem=recv_sems.at[NCH + len(rets)],
                    device_id=peer, device_id_type=pl.DeviceIdType.MESH)
                rr.start()
                rets.append(rr)

        rets = []
        wcopies[0][0].wait()
        wcopies[0][1].wait()
        own_ffn(0)
        recv_ffn(0)
        recv_ffn(1)
        wcopies[1][0].wait()
        wcopies[1][1].wait()
        recv_ffn(2)
        recv_ffn(3, splits=2)
        own_ffn(1)

        acc = lax.dot_general(
            sel_ref[pl.ds(own_base, E_LOCAL * GC)],
            y_ref[pl.ds(own_base, E_LOCAL * GC)],
            dimension_numbers=CONTRACT0,
            preferred_element_type=jnp.float32)

        for rr in rets:
            rr.wait_recv()

        obuf[:, :] = acc + lax.dot_general(
            sel_ref[pl.ds(peer_base, E_LOCAL * GC)],
            y_ref[pl.ds(peer_base, E_LOCAL * GC)],
            dimension_numbers=CONTRACT0,
            preferred_element_type=jnp.float32)
        ocopy = pltpu.make_async_copy(obuf, out_ref, out_sem)
        ocopy.start()

        for r in rs + rets:
            r.wait_send()
        ocopy.wait()

    return pl.pallas_call(
        body,
        out_shape=jax.ShapeDtypeStruct((T, D), jnp.float32),
        in_specs=[
            pl.BlockSpec(memory_space=pltpu.VMEM),
            pl.BlockSpec(memory_space=pltpu.VMEM),
            pl.BlockSpec(memory_space=pl.ANY),
            pl.BlockSpec(memory_space=pl.ANY),
        ],
        out_specs=pl.BlockSpec(memory_space=pl.ANY),
        scratch_shapes=[
            pltpu.VMEM((E * GC, D), jnp.bfloat16),
            pltpu.VMEM((E * GC, D), jnp.bfloat16),
            pltpu.VMEM((NCH, HC, D), jnp.bfloat16),
            pltpu.VMEM((NCH, HC, D), jnp.bfloat16),
            pltpu.VMEM((E_LOCAL, D, F), jnp.float32),
            pltpu.VMEM((E_LOCAL, F, D), jnp.float32),
            pltpu.VMEM((E * GC, T), jnp.bfloat16),
            pltpu.VMEM((T, D), jnp.float32),
            pltpu.SemaphoreType.DMA((2 * NCH + 1,)),
            pltpu.SemaphoreType.DMA((2 * NCH + 1,)),
            pltpu.SemaphoreType.DMA((2 * E_LOCAL,)),
            pltpu.SemaphoreType.DMA,
        ],
        compiler_params=pltpu.CompilerParams(
            collective_id=0, vmem_limit_bytes=60 * 1024 * 1024),
    )(x, a2d, W1, W2)
